# baseline (speedup 1.0000x reference)
import functools

import jax
import jax.numpy as jnp
from jax import lax
from jax.experimental import pallas as pl
from jax.experimental.pallas import tpu as pltpu
from jax.experimental.pallas import tpu_sc as plsc


def _gather_kernel(total, D, num_workers, chunk, nbuf):
    per_w = total // num_workers
    n_chunks = per_w // chunk
    mesh = plsc.VectorSubcoreMesh(core_axis_name="c", subcore_axis_name="s")
    n_groups = n_chunks // nbuf

    @functools.partial(
        pl.kernel,
        mesh=mesh,
        out_type=jax.ShapeDtypeStruct((total, D), jnp.float32),
        scratch_types=[
            pltpu.VMEM((per_w,), jnp.int32),
            pltpu.VMEM((nbuf, chunk, D), jnp.float32),
            [pltpu.SemaphoreType.DMA] * nbuf,
            [pltpu.SemaphoreType.DMA] * nbuf,
        ],
        compiler_params=pltpu.CompilerParams(use_tc_tiling_on_sc=False),
    )
    def k(idx_hbm, table_hbm, out_hbm, idx_v, rows_v, gsems, osems):
        nc = lax.axis_size("c")
        wid = lax.axis_index("s") * nc + lax.axis_index("c")
        base = wid * per_w

        pltpu.sync_copy(idx_hbm.at[pl.ds(base, per_w)], idx_v)

        def start_gather(c, b):
            return pltpu.async_copy(
                table_hbm.at[idx_v.at[pl.ds(c * chunk, chunk)]],
                rows_v.at[b],
                gsems[b],
            )

        def wait_gather(b):
            pltpu.make_async_copy(
                table_hbm.at[idx_v.at[pl.ds(0, chunk)]], rows_v.at[b], gsems[b]
            ).wait()

        def start_out(c, b):
            return pltpu.async_copy(
                rows_v.at[b],
                out_hbm.at[pl.ds(base + c * chunk, chunk)],
                osems[b],
            )

        def wait_out(b):
            pltpu.make_async_copy(
                rows_v.at[b], out_hbm.at[pl.ds(base, chunk)], osems[b]
            ).wait()

        for b in range(nbuf):
            start_gather(b, b)

        def body(g, carry):
            for b in range(nbuf):
                wait_gather(b)
                start_out(g * nbuf + b, b)
            for b in range(nbuf):
                wait_out(b)
                start_gather((g + 1) * nbuf + b, b)
            return carry

        lax.fori_loop(0, n_groups - 1, body, 0)

        for b in range(nbuf):
            wait_gather(b)
            start_out((n_groups - 1) * nbuf + b, b)
        for b in range(nbuf):
            wait_out(b)

    return k


def kernel(indexs, weights):
    B, H = indexs.shape
    V, D = weights.shape
    total = B * H
    idx_flat = indexs.reshape(total).astype(jnp.int32)
    out = _gather_kernel(total, D, 32, 256, 10)(idx_flat, weights)
    return out.reshape(B, H, D)


# speedup vs baseline: 1.5900x; 1.5900x over previous
import functools

import jax
import jax.numpy as jnp
from jax import lax
from jax.experimental import pallas as pl
from jax.experimental.pallas import tpu as pltpu
from jax.experimental.pallas import tpu_sc as plsc


def _gather_kernel(B, H, D, num_workers, cb):
    bat_w = B // num_workers
    n_chunks = bat_w // cb
    mesh = plsc.VectorSubcoreMesh(core_axis_name="c", subcore_axis_name="s")

    @functools.partial(
        pl.kernel,
        mesh=mesh,
        out_type=jax.ShapeDtypeStruct((B, H, D), jnp.float32),
        scratch_types=[
            pltpu.VMEM((bat_w, H), jnp.int32),
            pltpu.VMEM((cb, H, D), jnp.float32),
            pltpu.SemaphoreType.DMA,
        ],
        compiler_params=pltpu.CompilerParams(use_tc_tiling_on_sc=False),
    )
    def k(idx_hbm, table_hbm, out_hbm, idx_v, rows_v, sem):
        nc = lax.axis_size("c")
        wid = lax.axis_index("s") * nc + lax.axis_index("c")
        bbase = wid * bat_w
        pltpu.sync_copy(idx_hbm.at[pl.ds(bbase, bat_w)], idx_v)

        def body(c, carry):
            for j in range(cb):
                pltpu.async_copy(
                    table_hbm.at[idx_v.at[c * cb + j]], rows_v.at[j], sem
                )
            for j in range(cb):
                pltpu.make_async_copy(
                    table_hbm.at[idx_v.at[0]], rows_v.at[j], sem
                ).wait()
            pltpu.sync_copy(rows_v, out_hbm.at[pl.ds(bbase + c * cb, cb)])
            return carry

        lax.fori_loop(0, n_chunks, body, 0)

    return k


def kernel(indexs, weights):
    B, H = indexs.shape
    V, D = weights.shape
    out = _gather_kernel(B, H, D, 32, 16)(indexs.astype(jnp.int32), weights)
    return out
